# baseline (device time: 90468 ns/iter reference)
import jax
import jax.numpy as jnp
from jax import lax
from jax.experimental import pallas as pl
from jax.experimental.pallas import tpu as pltpu

N_DEV = 32


def kernel(x, w_mat, scale_x, scale_w):
    k_dim, k_per = x.shape
    n_dim = w_mat.shape[1]
    m_per = k_dim // N_DEV

    def body(x_ref, w_ref, sx_ref, sw_ref, out_ref, xg_ref, send_sems, recv_sems):
        kstep = pl.program_id(0)
        me = lax.axis_index("i")

        @pl.when(kstep == 0)
        def _issue_a2a():
            for d in range(1, N_DEV):
                dst = lax.rem(me + d, N_DEV)
                pltpu.make_async_remote_copy(
                    src_ref=x_ref.at[pl.ds(dst * m_per, m_per), :],
                    dst_ref=xg_ref.at[me],
                    send_sem=send_sems.at[dst],
                    recv_sem=recv_sems.at[me],
                    device_id=(dst,),
                    device_id_type=pl.DeviceIdType.MESH,
                ).start()

        @pl.when(kstep != me)
        def _wait_recv():
            pltpu.make_async_remote_copy(
                src_ref=x_ref.at[pl.ds(0, m_per), :],
                dst_ref=xg_ref.at[kstep],
                send_sem=send_sems.at[kstep],
                recv_sem=recv_sems.at[kstep],
                device_id=(me,),
                device_id_type=pl.DeviceIdType.MESH,
            ).wait_recv()

        a_own = x_ref[pl.ds(me * m_per, m_per), :]
        a_rcv = xg_ref[kstep]
        a = jnp.where(kstep == me, a_own, a_rcv).astype(jnp.bfloat16)
        partial = jnp.dot(
            a, w_ref[...].astype(jnp.bfloat16),
            preferred_element_type=jnp.float32,
        )

        @pl.when(kstep == 0)
        def _init():
            out_ref[...] = partial

        @pl.when(kstep != 0)
        def _acc():
            out_ref[...] += partial

        @pl.when(kstep == N_DEV - 1)
        def _epilogue():
            for d in range(1, N_DEV):
                dst = lax.rem(me + d, N_DEV)
                pltpu.make_async_remote_copy(
                    src_ref=x_ref.at[pl.ds(dst * m_per, m_per), :],
                    dst_ref=xg_ref.at[me],
                    send_sem=send_sems.at[dst],
                    recv_sem=recv_sems.at[dst],
                    device_id=(dst,),
                    device_id_type=pl.DeviceIdType.MESH,
                ).wait_send()
            s = sx_ref[0] * sw_ref[0]
            y = out_ref[...] * s
            out_ref[...] = y / (1.0 + jnp.exp(-jnp.clip(y, -60.0, 60.0)))

    return pl.pallas_call(
        body,
        grid=(N_DEV,),
        out_shape=jax.ShapeDtypeStruct((m_per, n_dim), jnp.float32),
        in_specs=[
            pl.BlockSpec((k_dim, k_per), lambda k: (0, 0)),
            pl.BlockSpec((k_per, n_dim), lambda k: (k, 0)),
            pl.BlockSpec(memory_space=pltpu.SMEM),
            pl.BlockSpec(memory_space=pltpu.SMEM),
        ],
        out_specs=pl.BlockSpec((m_per, n_dim), lambda k: (0, 0)),
        scratch_shapes=[
            pltpu.VMEM((N_DEV, m_per, k_per), jnp.float32),
            pltpu.SemaphoreType.DMA((N_DEV,)),
            pltpu.SemaphoreType.DMA((N_DEV,)),
        ],
        compiler_params=pltpu.CompilerParams(
            dimension_semantics=("arbitrary",),
        ),
    )(x, w_mat, scale_x, scale_w)
